# Initial kernel scaffold; baseline (speedup 1.0000x reference)
import jax
import jax.numpy as jnp
from jax.experimental import pallas as pl


def kernel(edge_index, user_emb, item_emb):
    raise NotImplementedError("write your pallas kernel here")


# speedup vs baseline: 7.5242x; 7.5242x over previous
import functools

import jax
import jax.numpy as jnp
from jax import lax
from jax.experimental import pallas as pl
from jax.experimental.pallas import tpu as pltpu
from jax.experimental.pallas import tpu_sc as plsc

_NUM_USERS = 25000
_NUM_ITEMS = 25000
_N_NODES = _NUM_USERS + _NUM_ITEMS
_EMB = 64
_HALF = 32
_N_EDGES = 800000

_NTILES = 16
_B = 128

_RPT = 3200
_NPAD = _NTILES * _RPT
_WCH = _RPT // _B

_EPT = 50176
_ECH = _EPT // _B
_E_PAD = _NTILES * _EPT
_TRASH = _N_NODES


def _scband_body(xh, rowp, colp, out, ya, yb,
                 acc, cnt, dinv_t,
                 zc, ones,
                 ybuf, zbuf, obuf, gbuf, rbuf, cbuf, sem):
    s = lax.axis_index("s")
    c = lax.axis_index("c")
    coff = c * _NPAD
    rbase = s * _RPT
    ebase = s * _EPT

    f32 = jnp.float32
    z16 = jnp.zeros((16,), f32)
    one16 = jnp.ones((16,), f32)

    for j in range(_B // 16):
        zc[pl.ds(16 * j, 16)] = z16
        ones[pl.ds(16 * j, 16)] = one16

    @pl.loop(0, _WCH)
    def _(k):
        pltpu.sync_copy(zc, cnt.at[pl.ds(rbase + k * _B, _B)])

    plsc.subcore_barrier()

    @pl.loop(0, _ECH)
    def _(k):
        pltpu.sync_copy(rowp.at[pl.ds(ebase + k * _B, _B)], rbuf)
        pltpu.sync_copy(ones, cnt.at[rbuf], add=True)

    plsc.subcore_barrier()

    pltpu.sync_copy(cnt.at[pl.ds(rbase, _RPT)], dinv_t)

    @pl.loop(0, _RPT // 16)
    def _(i):
        d = jnp.maximum(dinv_t[pl.ds(i * 16, 16)], 1.0)
        bits = lax.bitcast_convert_type(d, jnp.int32)
        y = lax.bitcast_convert_type(
            0x5F3759DF - lax.shift_right_logical(bits, 1), f32)
        y = y * (1.5 - 0.5 * d * y * y)
        y = y * (1.5 - 0.5 * d * y * y)
        y = y * (1.5 - 0.5 * d * y * y)
        dinv_t[pl.ds(i * 16, 16)] = y

    @pl.loop(0, _WCH)
    def _(k):
        a = coff + rbase + k * _B
        pltpu.sync_copy(xh.at[pl.ds(a, _B)], gbuf)

        @pl.loop(0, _B // 16)
        def _(g):
            sv = dinv_t[pl.ds(k * _B + g * 16, 16)]
            for i in range(16):
                r = g * 16 + i
                ybuf[r, pl.ds(0, 16)] = gbuf[r, pl.ds(0, 16)] * sv[i]
                ybuf[r, pl.ds(16, 16)] = gbuf[r, pl.ds(16, 16)] * sv[i]

        pltpu.sync_copy(gbuf, out.at[pl.ds(a, _B)])
        pltpu.sync_copy(ybuf, ya.at[pl.ds(a, _B)])

    plsc.subcore_barrier()

    for layer, (ysrc, ydst) in enumerate(((ya, yb), (yb, ya), (ya, None))):
        last = ydst is None

        @pl.loop(0, _B)
        def _(i):
            zbuf[i, pl.ds(0, 16)] = z16
            zbuf[i, pl.ds(16, 16)] = z16

        @pl.loop(0, _WCH)
        def _(k):
            pltpu.sync_copy(zbuf, acc.at[pl.ds(rbase + k * _B, _B)])

        plsc.subcore_barrier()

        @pl.loop(0, _ECH)
        def _(k):
            e0 = ebase + k * _B
            pltpu.sync_copy(rowp.at[pl.ds(e0, _B)], rbuf)
            pltpu.sync_copy(colp.at[pl.ds(e0, _B)], cbuf)
            for j in range(_B // 16):
                cbuf[pl.ds(16 * j, 16)] = cbuf[pl.ds(16 * j, 16)] + coff
            pltpu.async_copy(ysrc.at[cbuf], gbuf, sem).wait()
            pltpu.sync_copy(gbuf, acc.at[rbuf], add=True)

        plsc.subcore_barrier()

        @pl.loop(0, _WCH)
        def _(k):
            a = coff + rbase + k * _B
            pltpu.sync_copy(acc.at[pl.ds(rbase + k * _B, _B)], zbuf)
            pltpu.sync_copy(out.at[pl.ds(a, _B)], obuf)

            @pl.loop(0, _B // 16)
            def _(g):
                sv = dinv_t[pl.ds(k * _B + g * 16, 16)]
                s2v = sv * sv
                for i in range(16):
                    r = g * 16 + i
                    zlo = zbuf[r, pl.ds(0, 16)]
                    zhi = zbuf[r, pl.ds(16, 16)]
                    olo = obuf[r, pl.ds(0, 16)] + sv[i] * zlo
                    ohi = obuf[r, pl.ds(16, 16)] + sv[i] * zhi
                    if last:
                        olo = olo * 0.25
                        ohi = ohi * 0.25
                    obuf[r, pl.ds(0, 16)] = olo
                    obuf[r, pl.ds(16, 16)] = ohi
                    if not last:
                        ybuf[r, pl.ds(0, 16)] = s2v[i] * zlo
                        ybuf[r, pl.ds(16, 16)] = s2v[i] * zhi

            pltpu.sync_copy(obuf, out.at[pl.ds(a, _B)])
            if not last:
                pltpu.sync_copy(ybuf, ydst.at[pl.ds(a, _B)])

        plsc.subcore_barrier()


@jax.jit
def kernel(edge_index, user_emb, item_emb):
    f32 = jnp.float32
    all_emb = jnp.concatenate([user_emb, item_emb], axis=0)
    pad = jnp.zeros((_NPAD - _N_NODES, _HALF), f32)
    xh = jnp.concatenate(
        [all_emb[:, :_HALF], pad, all_emb[:, _HALF:], pad], axis=0)

    npad_e = _E_PAD - _N_EDGES
    rowp = jnp.concatenate(
        [edge_index[0], jnp.full((npad_e,), _TRASH, jnp.int32)])
    colp = jnp.concatenate([edge_index[1], jnp.zeros((npad_e,), jnp.int32)])

    mesh = plsc.VectorSubcoreMesh(core_axis_name="c", subcore_axis_name="s")
    tbl = jax.ShapeDtypeStruct((2 * _NPAD, _HALF), f32)
    run = pl.kernel(
        _scband_body,
        out_type=(tbl, tbl, tbl),
        mesh=mesh,
        compiler_params=pltpu.CompilerParams(use_tc_tiling_on_sc=False),
        scratch_types=[
            pltpu.VMEM_SHARED((_NPAD, _HALF), f32),
            pltpu.VMEM_SHARED((_NPAD,), f32),
            pltpu.VMEM((_RPT,), f32),
            pltpu.VMEM((_B,), f32),
            pltpu.VMEM((_B,), f32),
            pltpu.VMEM((_B, _HALF), f32),
            pltpu.VMEM((_B, _HALF), f32),
            pltpu.VMEM((_B, _HALF), f32),
            pltpu.VMEM((_B, _HALF), f32),
            pltpu.VMEM((_B,), jnp.int32),
            pltpu.VMEM((_B,), jnp.int32),
            pltpu.SemaphoreType.DMA,
        ],
    )
    out, _, _ = run(xh, rowp, colp)

    final = jnp.concatenate(
        [out[:_N_NODES], out[_NPAD:_NPAD + _N_NODES]], axis=1)
    return (final[:_NUM_USERS], final[_NUM_USERS:])


# speedup vs baseline: 18.7506x; 2.4920x over previous
import jax
import jax.numpy as jnp
from jax import lax
from jax.experimental import pallas as pl
from jax.experimental.pallas import tpu as pltpu
from jax.experimental.pallas import tpu_sc as plsc

_NUM_USERS = 25000
_NUM_ITEMS = 25000
_N_NODES = _NUM_USERS + _NUM_ITEMS
_HALF = 32
_N_EDGES = 800000

_NTILES = 16
_B = 128
_SCH = 8
_M = 49
_EPT = _M * _SCH * _B
_E_PAD = _NTILES * _EPT
_EROWS = _E_PAD // _B

_RPT = 3136
_NPAD = _NTILES * _RPT
_WB = 112
_WCH = _RPT // _WB
_TRASH = _N_NODES


def _scband_body(xh, rowp, colp, out, ya, yb,
                 acc, cnt, dinv_t, zc, ones,
                 ri, ci, g, ybuf, zbuf, obuf,
                 sem_i, sg0, sg1, ss0, ss1, ss2, ss3):
    s = lax.axis_index("s")
    c = lax.axis_index("c")
    coff = c * _NPAD
    rbase = s * _RPT
    erow = s * (_EPT // _B)

    f32 = jnp.float32
    z16 = jnp.zeros((16,), f32)
    one16 = jnp.ones((16,), f32)
    sg = (sg0, sg1)
    ss = (ss0, ss1, ss2, ss3)

    for j in range(_WB // 16):
        zc[pl.ds(16 * j, 16)] = z16
    for j in range(_B // 16):
        ones[pl.ds(16 * j, 16)] = one16

    @pl.loop(0, _WCH)
    def _(k):
        pltpu.sync_copy(zc, cnt.at[pl.ds(rbase + k * _WB, _WB)])

    plsc.subcore_barrier()

    pltpu.sync_copy(rowp.at[pl.ds(erow, _SCH)], ri.at[pl.ds(0, _SCH)])

    @pl.loop(0, _M)
    def _(m):
        b = (m % 2) * _SCH
        bn = ((m + 1) % 2) * _SCH
        for j in range(_SCH):
            if j >= 4:
                pltpu.make_async_copy(
                    ones, cnt.at[ri.at[b + j - 4]], ss[j % 4]).wait()
            else:
                @pl.when(m > 0)
                def _():
                    pltpu.make_async_copy(
                        ones, cnt.at[ri.at[b + j]], ss[j % 4]).wait()
            if j == 3:
                @pl.when(m < _M - 1)
                def _():
                    pltpu.async_copy(
                        rowp.at[pl.ds(erow + (m + 1) * _SCH, _SCH)],
                        ri.at[pl.ds(bn, _SCH)], sem_i)
            if j == 7:
                @pl.when(m < _M - 1)
                def _():
                    pltpu.make_async_copy(
                        rowp.at[pl.ds(erow + (m + 1) * _SCH, _SCH)],
                        ri.at[pl.ds(bn, _SCH)], sem_i).wait()
            pltpu.async_copy(ones, cnt.at[ri.at[b + j]], ss[j % 4],
                             add=True)

    for t in range(4):
        pltpu.make_async_copy(ones, cnt.at[ri.at[t]], ss[t]).wait()

    plsc.subcore_barrier()

    pltpu.sync_copy(cnt.at[pl.ds(rbase, _RPT)], dinv_t)

    @pl.loop(0, _RPT // 16)
    def _(i):
        d = jnp.maximum(dinv_t[pl.ds(i * 16, 16)], 1.0)
        bits = lax.bitcast_convert_type(d, jnp.int32)
        y = lax.bitcast_convert_type(
            0x5F3759DF - lax.shift_right_logical(bits, 1), f32)
        y = y * (1.5 - 0.5 * d * y * y)
        y = y * (1.5 - 0.5 * d * y * y)
        y = y * (1.5 - 0.5 * d * y * y)
        dinv_t[pl.ds(i * 16, 16)] = y

    @pl.loop(0, _WCH)
    def _(k):
        a = coff + rbase + k * _WB
        pltpu.sync_copy(xh.at[pl.ds(a, _WB)], obuf)

        @pl.loop(0, _WB // 16)
        def _(gr):
            sv = dinv_t[pl.ds(k * _WB + gr * 16, 16)]
            for i in range(16):
                r = gr * 16 + i
                ybuf[r, pl.ds(0, 16)] = obuf[r, pl.ds(0, 16)] * sv[i]
                ybuf[r, pl.ds(16, 16)] = obuf[r, pl.ds(16, 16)] * sv[i]

        pltpu.sync_copy(obuf, out.at[pl.ds(a, _WB)])
        pltpu.sync_copy(ybuf, ya.at[pl.ds(a, _WB)])

    plsc.subcore_barrier()

    for layer, (ysrc, ydst) in enumerate(((ya, yb), (yb, ya), (ya, None))):
        last = ydst is None

        @pl.loop(0, _WB)
        def _(i):
            zbuf[i, pl.ds(0, 16)] = z16
            zbuf[i, pl.ds(16, 16)] = z16

        @pl.loop(0, _WCH)
        def _(k):
            pltpu.sync_copy(zbuf, acc.at[pl.ds(rbase + k * _WB, _WB)])

        plsc.subcore_barrier()

        def _adjust(base):
            for r in range(_SCH):
                for i in range(_B // 16):
                    ci[base + r, pl.ds(16 * i, 16)] = (
                        ci[base + r, pl.ds(16 * i, 16)] + coff)

        pltpu.sync_copy(rowp.at[pl.ds(erow, _SCH)], ri.at[pl.ds(0, _SCH)])
        pltpu.sync_copy(colp.at[pl.ds(erow, _SCH)], ci.at[pl.ds(0, _SCH)])
        _adjust(0)
        pltpu.async_copy(ysrc.at[ci.at[0]], g.at[0], sg[0])

        @pl.loop(0, _M)
        def _(m):
            b = (m % 2) * _SCH
            bn = ((m + 1) % 2) * _SCH
            for j in range(_SCH):
                if j == 0:
                    @pl.when(m > 0)
                    def _():
                        pltpu.make_async_copy(
                            g.at[1], acc.at[ri.at[b + 7]],
                            ss[1]).wait()
                else:
                    pltpu.make_async_copy(
                        g.at[(j + 1) % 2], acc.at[ri.at[b + j - 1]],
                        ss[(j + 1) % 2]).wait()
                if j == 2:
                    @pl.when(m < _M - 1)
                    def _():
                        pltpu.async_copy(
                            rowp.at[pl.ds(erow + (m + 1) * _SCH, _SCH)],
                            ri.at[pl.ds(bn, _SCH)], sem_i)
                        pltpu.async_copy(
                            colp.at[pl.ds(erow + (m + 1) * _SCH, _SCH)],
                            ci.at[pl.ds(bn, _SCH)], sem_i)
                if j == 7:
                    @pl.when(m < _M - 1)
                    def _():
                        pltpu.make_async_copy(
                            rowp.at[pl.ds(erow + (m + 1) * _SCH, _SCH)],
                            ri.at[pl.ds(bn, _SCH)], sem_i).wait()
                        pltpu.make_async_copy(
                            colp.at[pl.ds(erow + (m + 1) * _SCH, _SCH)],
                            ci.at[pl.ds(bn, _SCH)], sem_i).wait()
                        _adjust(bn)
                if j < 7:
                    pltpu.async_copy(ysrc.at[ci.at[b + j + 1]],
                                     g.at[(j + 1) % 2], sg[(j + 1) % 2])
                else:
                    @pl.when(m < _M - 1)
                    def _():
                        pltpu.async_copy(ysrc.at[ci.at[bn]],
                                         g.at[0], sg[0])
                pltpu.make_async_copy(ysrc.at[ci.at[b + j]],
                                      g.at[j % 2], sg[j % 2]).wait()
                pltpu.async_copy(g.at[j % 2], acc.at[ri.at[b + j]],
                                 ss[j % 2], add=True)

        pltpu.make_async_copy(g.at[1], acc.at[ri.at[1]], ss[1]).wait()

        plsc.subcore_barrier()

        @pl.loop(0, _WCH)
        def _(k):
            a = coff + rbase + k * _WB
            pltpu.sync_copy(acc.at[pl.ds(rbase + k * _WB, _WB)], zbuf)
            pltpu.sync_copy(out.at[pl.ds(a, _WB)], obuf)

            @pl.loop(0, _WB // 16)
            def _(gr):
                sv = dinv_t[pl.ds(k * _WB + gr * 16, 16)]
                s2v = sv * sv
                for i in range(16):
                    r = gr * 16 + i
                    zlo = zbuf[r, pl.ds(0, 16)]
                    zhi = zbuf[r, pl.ds(16, 16)]
                    olo = obuf[r, pl.ds(0, 16)] + sv[i] * zlo
                    ohi = obuf[r, pl.ds(16, 16)] + sv[i] * zhi
                    if last:
                        olo = olo * 0.25
                        ohi = ohi * 0.25
                    obuf[r, pl.ds(0, 16)] = olo
                    obuf[r, pl.ds(16, 16)] = ohi
                    if not last:
                        ybuf[r, pl.ds(0, 16)] = s2v[i] * zlo
                        ybuf[r, pl.ds(16, 16)] = s2v[i] * zhi

            pltpu.sync_copy(obuf, out.at[pl.ds(a, _WB)])
            if not last:
                pltpu.sync_copy(ybuf, ydst.at[pl.ds(a, _WB)])

        plsc.subcore_barrier()


@jax.jit
def kernel(edge_index, user_emb, item_emb):
    f32 = jnp.float32
    all_emb = jnp.concatenate([user_emb, item_emb], axis=0)
    pad = jnp.zeros((_NPAD - _N_NODES, _HALF), f32)
    xh = jnp.concatenate(
        [all_emb[:, :_HALF], pad, all_emb[:, _HALF:], pad], axis=0)

    npad_e = _E_PAD - _N_EDGES
    rowp = jnp.concatenate(
        [edge_index[0], jnp.full((npad_e,), _TRASH, jnp.int32)])
    colp = jnp.concatenate([edge_index[1], jnp.zeros((npad_e,), jnp.int32)])
    rowp = rowp.reshape(_EROWS, _B)
    colp = colp.reshape(_EROWS, _B)

    mesh = plsc.VectorSubcoreMesh(core_axis_name="c", subcore_axis_name="s")
    tbl = jax.ShapeDtypeStruct((2 * _NPAD, _HALF), f32)
    run = pl.kernel(
        _scband_body,
        out_type=(tbl, tbl, tbl),
        mesh=mesh,
        compiler_params=pltpu.CompilerParams(use_tc_tiling_on_sc=False),
        scratch_types=[
            pltpu.VMEM_SHARED((_NPAD, _HALF), f32),
            pltpu.VMEM_SHARED((_NPAD,), f32),
            pltpu.VMEM((_RPT,), f32),
            pltpu.VMEM((_WB,), f32),
            pltpu.VMEM((_B,), f32),
            pltpu.VMEM((2 * _SCH, _B), jnp.int32),
            pltpu.VMEM((2 * _SCH, _B), jnp.int32),
            pltpu.VMEM((2, _B, _HALF), f32),
            pltpu.VMEM((_WB, _HALF), f32),
            pltpu.VMEM((_WB, _HALF), f32),
            pltpu.VMEM((_WB, _HALF), f32),
            pltpu.SemaphoreType.DMA,
            pltpu.SemaphoreType.DMA,
            pltpu.SemaphoreType.DMA,
            pltpu.SemaphoreType.DMA,
            pltpu.SemaphoreType.DMA,
            pltpu.SemaphoreType.DMA,
            pltpu.SemaphoreType.DMA,
        ],
    )
    out, _, _ = run(xh, rowp, colp)

    final = jnp.concatenate(
        [out[:_N_NODES], out[_NPAD:_NPAD + _N_NODES]], axis=1)
    return (final[:_NUM_USERS], final[_NUM_USERS:])


# speedup vs baseline: 21.0747x; 1.1239x over previous
import jax
import jax.numpy as jnp
from jax import lax
from jax.experimental import pallas as pl
from jax.experimental.pallas import tpu as pltpu
from jax.experimental.pallas import tpu_sc as plsc

_NUM_USERS = 25000
_NUM_ITEMS = 25000
_N_NODES = _NUM_USERS + _NUM_ITEMS
_HALF = 32
_N_EDGES = 800000

_NTILES = 16
_B = 128
_SCH = 8
_M = 49
_EPT = _M * _SCH * _B
_E_PAD = _NTILES * _EPT
_EROWS = _E_PAD // _B

_RPT = 3136
_NPAD = _NTILES * _RPT
_WB = 64
_WCH = _RPT // _WB
_TRASH = _N_NODES


def _scband_body(xh, rowp, colp, out, ya, yb,
                 acc, cnt, zc, ones, dv,
                 ri, ci, g, ybuf, zbuf, obuf,
                 sem_i, sg0, sg1, sg2, sg3, ss0, ss1, ss2, ss3):
    s = lax.axis_index("s")
    c = lax.axis_index("c")
    coff = c * _NPAD
    rbase = s * _RPT
    erow = s * (_EPT // _B)

    f32 = jnp.float32
    z16 = jnp.zeros((16,), f32)
    one16 = jnp.ones((16,), f32)
    sg = (sg0, sg1, sg2, sg3)
    ss = (ss0, ss1, ss2, ss3)

    for j in range(_WB // 16):
        zc[pl.ds(16 * j, 16)] = z16
    for j in range(_B // 16):
        ones[pl.ds(16 * j, 16)] = one16

    @pl.loop(0, _WCH)
    def _(k):
        pltpu.sync_copy(zc, cnt.at[pl.ds(rbase + k * _WB, _WB)])

    plsc.subcore_barrier()

    pltpu.sync_copy(rowp.at[pl.ds(erow, _SCH)], ri.at[pl.ds(0, _SCH)])

    @pl.loop(0, _M)
    def _(m):
        b = (m % 2) * _SCH
        bn = ((m + 1) % 2) * _SCH
        for j in range(_SCH):
            if j >= 4:
                pltpu.make_async_copy(
                    ones, cnt.at[ri.at[b + j - 4]], ss[j % 4]).wait()
            else:
                @pl.when(m > 0)
                def _():
                    pltpu.make_async_copy(
                        ones, cnt.at[ri.at[b + j]], ss[j % 4]).wait()
            if j == 3:
                @pl.when(m < _M - 1)
                def _():
                    pltpu.async_copy(
                        rowp.at[pl.ds(erow + (m + 1) * _SCH, _SCH)],
                        ri.at[pl.ds(bn, _SCH)], sem_i)
            if j == 7:
                @pl.when(m < _M - 1)
                def _():
                    pltpu.make_async_copy(
                        rowp.at[pl.ds(erow + (m + 1) * _SCH, _SCH)],
                        ri.at[pl.ds(bn, _SCH)], sem_i).wait()
            pltpu.async_copy(ones, cnt.at[ri.at[b + j]], ss[j % 4],
                             add=True)

    for t in range(4):
        pltpu.make_async_copy(ones, cnt.at[ri.at[t]], ss[t]).wait()

    plsc.subcore_barrier()

    @pl.loop(0, _WCH)
    def _(k):
        r0 = rbase + k * _WB
        a = coff + r0
        pltpu.sync_copy(xh.at[pl.ds(a, _WB)], obuf)
        pltpu.sync_copy(cnt.at[pl.ds(r0, _WB)], dv)

        @pl.loop(0, _WB // 16)
        def _(gr):
            d = jnp.maximum(dv[pl.ds(gr * 16, 16)], 1.0)
            bits = lax.bitcast_convert_type(d, jnp.int32)
            y = lax.bitcast_convert_type(
                0x5F3759DF - lax.shift_right_logical(bits, 1), f32)
            y = y * (1.5 - 0.5 * d * y * y)
            y = y * (1.5 - 0.5 * d * y * y)
            y = y * (1.5 - 0.5 * d * y * y)
            dv[pl.ds(gr * 16, 16)] = y

        pltpu.sync_copy(dv, cnt.at[pl.ds(r0, _WB)])

        @pl.loop(0, _WB // 16)
        def _(gr):
            sv = dv[pl.ds(gr * 16, 16)]
            for i in range(16):
                r = gr * 16 + i
                ybuf[r, pl.ds(0, 16)] = obuf[r, pl.ds(0, 16)] * sv[i]
                ybuf[r, pl.ds(16, 16)] = obuf[r, pl.ds(16, 16)] * sv[i]

        pltpu.sync_copy(obuf, out.at[pl.ds(a, _WB)])
        pltpu.sync_copy(ybuf, ya.at[pl.ds(a, _WB)])

    plsc.subcore_barrier()

    for layer, (ysrc, ydst) in enumerate(((ya, yb), (yb, ya), (ya, None))):
        last = ydst is None

        @pl.loop(0, _WB)
        def _(i):
            zbuf[i, pl.ds(0, 16)] = z16
            zbuf[i, pl.ds(16, 16)] = z16

        @pl.loop(0, _WCH)
        def _(k):
            pltpu.sync_copy(zbuf, acc.at[pl.ds(rbase + k * _WB, _WB)])

        plsc.subcore_barrier()

        def _adjust(base):
            for r in range(_SCH):
                for i in range(_B // 16):
                    ci[base + r, pl.ds(16 * i, 16)] = (
                        ci[base + r, pl.ds(16 * i, 16)] + coff)

        pltpu.sync_copy(rowp.at[pl.ds(erow, _SCH)], ri.at[pl.ds(0, _SCH)])
        pltpu.sync_copy(colp.at[pl.ds(erow, _SCH)], ci.at[pl.ds(0, _SCH)])
        _adjust(0)
        for t in range(3):
            pltpu.async_copy(ysrc.at[ci.at[t]], g.at[t], sg[t])

        @pl.loop(0, _M)
        def _(m):
            b = (m % 2) * _SCH
            bn = ((m + 1) % 2) * _SCH
            for j in range(_SCH):
                if j == 1:
                    @pl.when(m < _M - 1)
                    def _():
                        pltpu.async_copy(
                            rowp.at[pl.ds(erow + (m + 1) * _SCH, _SCH)],
                            ri.at[pl.ds(bn, _SCH)], sem_i)
                        pltpu.async_copy(
                            colp.at[pl.ds(erow + (m + 1) * _SCH, _SCH)],
                            ci.at[pl.ds(bn, _SCH)], sem_i)
                if j == 4:
                    @pl.when(m < _M - 1)
                    def _():
                        pltpu.make_async_copy(
                            rowp.at[pl.ds(erow + (m + 1) * _SCH, _SCH)],
                            ri.at[pl.ds(bn, _SCH)], sem_i).wait()
                        pltpu.make_async_copy(
                            colp.at[pl.ds(erow + (m + 1) * _SCH, _SCH)],
                            ci.at[pl.ds(bn, _SCH)], sem_i).wait()
                        _adjust(bn)
                pltpu.make_async_copy(ysrc.at[ci.at[b + j]],
                                      g.at[j % 4], sg[j % 4]).wait()
                pltpu.async_copy(g.at[j % 4], acc.at[ri.at[b + j]],
                                 ss[j % 4], add=True)
                if j == 0:
                    @pl.when(m > 0)
                    def _():
                        pltpu.make_async_copy(
                            g.at[3], acc.at[ri.at[bn + 7]],
                            ss[3]).wait()
                else:
                    pltpu.make_async_copy(
                        g.at[(j + 3) % 4], acc.at[ri.at[b + j - 1]],
                        ss[(j + 3) % 4]).wait()
                if j < 5:
                    pltpu.async_copy(ysrc.at[ci.at[b + j + 3]],
                                     g.at[(j + 3) % 4], sg[(j + 3) % 4])
                else:
                    @pl.when(m < _M - 1)
                    def _():
                        pltpu.async_copy(ysrc.at[ci.at[bn + j - 5]],
                                         g.at[(j + 3) % 4],
                                         sg[(j + 3) % 4])

        pltpu.make_async_copy(g.at[3], acc.at[ri.at[7]], ss[3]).wait()

        plsc.subcore_barrier()

        @pl.loop(0, _WCH)
        def _(k):
            r0 = rbase + k * _WB
            a = coff + r0
            pltpu.sync_copy(acc.at[pl.ds(r0, _WB)], zbuf)
            pltpu.sync_copy(out.at[pl.ds(a, _WB)], obuf)
            pltpu.sync_copy(cnt.at[pl.ds(r0, _WB)], dv)

            @pl.loop(0, _WB // 16)
            def _(gr):
                sv = dv[pl.ds(gr * 16, 16)]
                s2v = sv * sv
                for i in range(16):
                    r = gr * 16 + i
                    zlo = zbuf[r, pl.ds(0, 16)]
                    zhi = zbuf[r, pl.ds(16, 16)]
                    olo = obuf[r, pl.ds(0, 16)] + sv[i] * zlo
                    ohi = obuf[r, pl.ds(16, 16)] + sv[i] * zhi
                    if last:
                        olo = olo * 0.25
                        ohi = ohi * 0.25
                    obuf[r, pl.ds(0, 16)] = olo
                    obuf[r, pl.ds(16, 16)] = ohi
                    if not last:
                        ybuf[r, pl.ds(0, 16)] = s2v[i] * zlo
                        ybuf[r, pl.ds(16, 16)] = s2v[i] * zhi

            pltpu.sync_copy(obuf, out.at[pl.ds(a, _WB)])
            if not last:
                pltpu.sync_copy(ybuf, ydst.at[pl.ds(a, _WB)])

        plsc.subcore_barrier()


@jax.jit
def kernel(edge_index, user_emb, item_emb):
    f32 = jnp.float32
    all_emb = jnp.concatenate([user_emb, item_emb], axis=0)
    pad = jnp.zeros((_NPAD - _N_NODES, _HALF), f32)
    xh = jnp.concatenate(
        [all_emb[:, :_HALF], pad, all_emb[:, _HALF:], pad], axis=0)

    npad_e = _E_PAD - _N_EDGES
    rowp = jnp.concatenate(
        [edge_index[0], jnp.full((npad_e,), _TRASH, jnp.int32)])
    colp = jnp.concatenate([edge_index[1], jnp.zeros((npad_e,), jnp.int32)])
    rowp = rowp.reshape(_EROWS, _B)
    colp = colp.reshape(_EROWS, _B)

    mesh = plsc.VectorSubcoreMesh(core_axis_name="c", subcore_axis_name="s")
    tbl = jax.ShapeDtypeStruct((2 * _NPAD, _HALF), f32)
    run = pl.kernel(
        _scband_body,
        out_type=(tbl, tbl, tbl),
        mesh=mesh,
        compiler_params=pltpu.CompilerParams(use_tc_tiling_on_sc=False),
        scratch_types=[
            pltpu.VMEM_SHARED((_NPAD, _HALF), f32),
            pltpu.VMEM_SHARED((_NPAD,), f32),
            pltpu.VMEM((_WB,), f32),
            pltpu.VMEM((_B,), f32),
            pltpu.VMEM((_WB,), f32),
            pltpu.VMEM((2 * _SCH, _B), jnp.int32),
            pltpu.VMEM((2 * _SCH, _B), jnp.int32),
            pltpu.VMEM((4, _B, _HALF), f32),
            pltpu.VMEM((_WB, _HALF), f32),
            pltpu.VMEM((_WB, _HALF), f32),
            pltpu.VMEM((_WB, _HALF), f32),
            pltpu.SemaphoreType.DMA,
            pltpu.SemaphoreType.DMA,
            pltpu.SemaphoreType.DMA,
            pltpu.SemaphoreType.DMA,
            pltpu.SemaphoreType.DMA,
            pltpu.SemaphoreType.DMA,
            pltpu.SemaphoreType.DMA,
            pltpu.SemaphoreType.DMA,
            pltpu.SemaphoreType.DMA,
        ],
    )
    out, _, _ = run(xh, rowp, colp)

    final = jnp.concatenate(
        [out[:_N_NODES], out[_NPAD:_NPAD + _N_NODES]], axis=1)
    return (final[:_NUM_USERS], final[_NUM_USERS:])


# speedup vs baseline: 21.1272x; 1.0025x over previous
import jax
import jax.numpy as jnp
from jax import lax
from jax.experimental import pallas as pl
from jax.experimental.pallas import tpu as pltpu
from jax.experimental.pallas import tpu_sc as plsc

_NUM_USERS = 25000
_NUM_ITEMS = 25000
_N_NODES = _NUM_USERS + _NUM_ITEMS
_HALF = 32
_N_EDGES = 800000

_NTILES = 16
_B = 128
_SCH = 8
_M = 49
_EPT = _M * _SCH * _B
_E_PAD = _NTILES * _EPT
_EROWS = _E_PAD // _B

_RPT = 3136
_NPAD = _NTILES * _RPT
_WB = 64
_WCH = _RPT // _WB
_TRASH = _N_NODES


def _scband_body(xh, rowp, colp, out, ya, yb,
                 acc, cnt, zc, ones, dv,
                 ri, ci, g, ybuf, zbuf, obuf,
                 sem_i, sg0, sg1, sg2, sg3, ss0, ss1, ss2, ss3):
    s = lax.axis_index("s")
    c = lax.axis_index("c")
    coff = c * _NPAD
    rbase = s * _RPT
    erow = s * (_EPT // _B)

    f32 = jnp.float32
    z16 = jnp.zeros((16,), f32)
    one16 = jnp.ones((16,), f32)
    sg = (sg0, sg1, sg2, sg3)
    ss = (ss0, ss1, ss2, ss3)

    for j in range(_WB // 16):
        zc[pl.ds(16 * j, 16)] = z16
    for j in range(_B // 16):
        ones[pl.ds(16 * j, 16)] = one16

    @pl.loop(0, _WCH)
    def _(k):
        pltpu.sync_copy(zc, cnt.at[pl.ds(rbase + k * _WB, _WB)])

    plsc.subcore_barrier()

    pltpu.sync_copy(rowp.at[pl.ds(erow, _SCH)], ri.at[pl.ds(0, _SCH)])

    @pl.loop(0, _M)
    def _(m):
        b = (m % 2) * _SCH
        bn = ((m + 1) % 2) * _SCH
        for j in range(_SCH):
            if j >= 4:
                pltpu.make_async_copy(
                    ones, cnt.at[ri.at[b + j - 4]], ss[j % 4]).wait()
            else:
                @pl.when(m > 0)
                def _():
                    pltpu.make_async_copy(
                        ones, cnt.at[ri.at[b + j]], ss[j % 4]).wait()
            if j == 3:
                @pl.when(m < _M - 1)
                def _():
                    pltpu.async_copy(
                        rowp.at[pl.ds(erow + (m + 1) * _SCH, _SCH)],
                        ri.at[pl.ds(bn, _SCH)], sem_i)
            if j == 7:
                @pl.when(m < _M - 1)
                def _():
                    pltpu.make_async_copy(
                        rowp.at[pl.ds(erow + (m + 1) * _SCH, _SCH)],
                        ri.at[pl.ds(bn, _SCH)], sem_i).wait()
            pltpu.async_copy(ones, cnt.at[ri.at[b + j]], ss[j % 4],
                             add=True)

    for t in range(4):
        pltpu.make_async_copy(ones, cnt.at[ri.at[t]], ss[t]).wait()

    plsc.subcore_barrier()

    @pl.loop(0, _WCH)
    def _(k):
        r0 = rbase + k * _WB
        a = coff + r0
        pltpu.sync_copy(xh.at[pl.ds(a, _WB)], obuf)
        pltpu.sync_copy(cnt.at[pl.ds(r0, _WB)], dv)

        @pl.loop(0, _WB // 16)
        def _(gr):
            d = jnp.maximum(dv[pl.ds(gr * 16, 16)], 1.0)
            bits = lax.bitcast_convert_type(d, jnp.int32)
            y = lax.bitcast_convert_type(
                0x5F3759DF - lax.shift_right_logical(bits, 1), f32)
            y = y * (1.5 - 0.5 * d * y * y)
            y = y * (1.5 - 0.5 * d * y * y)
            y = y * (1.5 - 0.5 * d * y * y)
            dv[pl.ds(gr * 16, 16)] = y

        pltpu.sync_copy(dv, cnt.at[pl.ds(r0, _WB)])

        @pl.loop(0, _WB // 16)
        def _(gr):
            sv = dv[pl.ds(gr * 16, 16)]
            for i in range(16):
                r = gr * 16 + i
                ybuf[r, pl.ds(0, 16)] = obuf[r, pl.ds(0, 16)] * sv[i]
                ybuf[r, pl.ds(16, 16)] = obuf[r, pl.ds(16, 16)] * sv[i]

        pltpu.sync_copy(obuf, out.at[pl.ds(a, _WB)])
        pltpu.sync_copy(ybuf, ya.at[pl.ds(a, _WB)])

    plsc.subcore_barrier()

    for layer, (ysrc, ydst) in enumerate(((ya, yb), (yb, ya), (ya, None))):
        last = ydst is None

        @pl.loop(0, _WB)
        def _(i):
            zbuf[i, pl.ds(0, 16)] = z16
            zbuf[i, pl.ds(16, 16)] = z16

        @pl.loop(0, _WCH)
        def _(k):
            pltpu.sync_copy(zbuf, acc.at[pl.ds(rbase + k * _WB, _WB)])

        plsc.subcore_barrier()

        cp = colp.at[c]
        pltpu.sync_copy(rowp.at[pl.ds(erow, _SCH)], ri.at[pl.ds(0, _SCH)])
        pltpu.sync_copy(cp.at[pl.ds(erow, _SCH)], ci.at[pl.ds(0, _SCH)])
        for t in range(3):
            pltpu.async_copy(ysrc.at[ci.at[t]], g.at[t], sg[t])

        @pl.loop(0, _M)
        def _(m):
            b = (m % 2) * _SCH
            bn = ((m + 1) % 2) * _SCH
            for j in range(_SCH):
                if j == 1:
                    @pl.when(m < _M - 1)
                    def _():
                        pltpu.async_copy(
                            rowp.at[pl.ds(erow + (m + 1) * _SCH, _SCH)],
                            ri.at[pl.ds(bn, _SCH)], sem_i)
                        pltpu.async_copy(
                            cp.at[pl.ds(erow + (m + 1) * _SCH, _SCH)],
                            ci.at[pl.ds(bn, _SCH)], sem_i)
                if j == 4:
                    @pl.when(m < _M - 1)
                    def _():
                        pltpu.make_async_copy(
                            rowp.at[pl.ds(erow + (m + 1) * _SCH, _SCH)],
                            ri.at[pl.ds(bn, _SCH)], sem_i).wait()
                        pltpu.make_async_copy(
                            cp.at[pl.ds(erow + (m + 1) * _SCH, _SCH)],
                            ci.at[pl.ds(bn, _SCH)], sem_i).wait()
                pltpu.make_async_copy(ysrc.at[ci.at[b + j]],
                                      g.at[j % 4], sg[j % 4]).wait()
                pltpu.async_copy(g.at[j % 4], acc.at[ri.at[b + j]],
                                 ss[j % 4], add=True)
                if j == 0:
                    @pl.when(m > 0)
                    def _():
                        pltpu.make_async_copy(
                            g.at[3], acc.at[ri.at[bn + 7]],
                            ss[3]).wait()
                else:
                    pltpu.make_async_copy(
                        g.at[(j + 3) % 4], acc.at[ri.at[b + j - 1]],
                        ss[(j + 3) % 4]).wait()
                if j < 5:
                    pltpu.async_copy(ysrc.at[ci.at[b + j + 3]],
                                     g.at[(j + 3) % 4], sg[(j + 3) % 4])
                else:
                    @pl.when(m < _M - 1)
                    def _():
                        pltpu.async_copy(ysrc.at[ci.at[bn + j - 5]],
                                         g.at[(j + 3) % 4],
                                         sg[(j + 3) % 4])

        pltpu.make_async_copy(g.at[3], acc.at[ri.at[7]], ss[3]).wait()

        plsc.subcore_barrier()

        @pl.loop(0, _WCH)
        def _(k):
            r0 = rbase + k * _WB
            a = coff + r0
            pltpu.sync_copy(acc.at[pl.ds(r0, _WB)], zbuf)
            pltpu.sync_copy(out.at[pl.ds(a, _WB)], obuf)
            pltpu.sync_copy(cnt.at[pl.ds(r0, _WB)], dv)

            @pl.loop(0, _WB // 16)
            def _(gr):
                sv = dv[pl.ds(gr * 16, 16)]
                s2v = sv * sv
                for i in range(16):
                    r = gr * 16 + i
                    zlo = zbuf[r, pl.ds(0, 16)]
                    zhi = zbuf[r, pl.ds(16, 16)]
                    olo = obuf[r, pl.ds(0, 16)] + sv[i] * zlo
                    ohi = obuf[r, pl.ds(16, 16)] + sv[i] * zhi
                    if last:
                        olo = olo * 0.25
                        ohi = ohi * 0.25
                    obuf[r, pl.ds(0, 16)] = olo
                    obuf[r, pl.ds(16, 16)] = ohi
                    if not last:
                        ybuf[r, pl.ds(0, 16)] = s2v[i] * zlo
                        ybuf[r, pl.ds(16, 16)] = s2v[i] * zhi

            pltpu.sync_copy(obuf, out.at[pl.ds(a, _WB)])
            if not last:
                pltpu.sync_copy(ybuf, ydst.at[pl.ds(a, _WB)])

        plsc.subcore_barrier()


@jax.jit
def kernel(edge_index, user_emb, item_emb):
    f32 = jnp.float32
    all_emb = jnp.concatenate([user_emb, item_emb], axis=0)
    pad = jnp.zeros((_NPAD - _N_NODES, _HALF), f32)
    xh = jnp.concatenate(
        [all_emb[:, :_HALF], pad, all_emb[:, _HALF:], pad], axis=0)

    npad_e = _E_PAD - _N_EDGES
    rowp = jnp.concatenate(
        [edge_index[0], jnp.full((npad_e,), _TRASH, jnp.int32)])
    colp = jnp.concatenate([edge_index[1], jnp.zeros((npad_e,), jnp.int32)])
    rowp = rowp.reshape(_EROWS, _B)
    colp = colp.reshape(_EROWS, _B)
    colp = jnp.stack([colp, colp + _NPAD])

    mesh = plsc.VectorSubcoreMesh(core_axis_name="c", subcore_axis_name="s")
    tbl = jax.ShapeDtypeStruct((2 * _NPAD, _HALF), f32)
    run = pl.kernel(
        _scband_body,
        out_type=(tbl, tbl, tbl),
        mesh=mesh,
        compiler_params=pltpu.CompilerParams(use_tc_tiling_on_sc=False),
        scratch_types=[
            pltpu.VMEM_SHARED((_NPAD, _HALF), f32),
            pltpu.VMEM_SHARED((_NPAD,), f32),
            pltpu.VMEM((_WB,), f32),
            pltpu.VMEM((_B,), f32),
            pltpu.VMEM((_WB,), f32),
            pltpu.VMEM((2 * _SCH, _B), jnp.int32),
            pltpu.VMEM((2 * _SCH, _B), jnp.int32),
            pltpu.VMEM((4, _B, _HALF), f32),
            pltpu.VMEM((_WB, _HALF), f32),
            pltpu.VMEM((_WB, _HALF), f32),
            pltpu.VMEM((_WB, _HALF), f32),
            pltpu.SemaphoreType.DMA,
            pltpu.SemaphoreType.DMA,
            pltpu.SemaphoreType.DMA,
            pltpu.SemaphoreType.DMA,
            pltpu.SemaphoreType.DMA,
            pltpu.SemaphoreType.DMA,
            pltpu.SemaphoreType.DMA,
            pltpu.SemaphoreType.DMA,
            pltpu.SemaphoreType.DMA,
        ],
    )
    out, _, _ = run(xh, rowp, colp)

    final = jnp.concatenate(
        [out[:_N_NODES], out[_NPAD:_NPAD + _N_NODES]], axis=1)
    return (final[:_NUM_USERS], final[_NUM_USERS:])
